# baseline (speedup 1.0000x reference)
import functools

import jax
import jax.numpy as jnp
from jax import lax
from jax.experimental import pallas as pl
from jax.experimental.pallas import tpu as pltpu
from jax.experimental.pallas import tpu_sc as plsc

BATCH = 16384
NUM_FEATURES = 100
NUM_BUCKETS = 20
EMBED_DIM = 32

NC = 2
NS = 16
NW = NC * NS

TBL = NUM_FEATURES * EMBED_DIM * NUM_BUCKETS
LANES = 16

FPAD = 104
TI = EMBED_DIM // 8
TJ = BATCH // 128
TJB = 16
NBLK = TJ // TJB
NGROUPS = NUM_FEATURES * TI
NBLOCKS = NGROUPS * NBLK


BM = 2048
NBM = BATCH // BM


def _minmax_body(x_ref, mm_ref):
    i = pl.program_id(0)
    xb = x_ref[...]
    mn = jnp.min(xb, axis=0, keepdims=True)
    mx = jnp.max(xb, axis=0, keepdims=True)

    @pl.when(i == 0)
    def _():
        mm_ref[0:1, :] = mn
        mm_ref[1:2, :] = mx

    @pl.when(i > 0)
    def _():
        mm_ref[0:1, :] = jnp.minimum(mm_ref[0:1, :], mn)
        mm_ref[1:2, :] = jnp.maximum(mm_ref[1:2, :], mx)


def _bucket_body(bs_ref, mm_ref, x_ref, out_ref):
    xb = x_ref[...]
    mn = mm_ref[0:1, :]
    mx = mm_ref[1:2, :]
    xn = (xb - mn) / (mx - mn + 1e-8)
    cnt = (xn > bs_ref[0]).astype(jnp.int32)
    for i in range(1, NUM_BUCKETS - 1):
        cnt += (xn > bs_ref[i]).astype(jnp.int32)
    bt = jnp.transpose(cnt)
    pad = jnp.zeros((FPAD - NUM_FEATURES, BM), jnp.int32)
    out_ref[...] = jnp.concatenate([bt, pad], axis=0)


def _bucket_t(x, b_sig):
    mm = pl.pallas_call(
        _minmax_body,
        grid=(NBM,),
        in_specs=[pl.BlockSpec((BM, NUM_FEATURES), lambda i: (i, 0))],
        out_specs=pl.BlockSpec((2, NUM_FEATURES), lambda i: (0, 0)),
        out_shape=jax.ShapeDtypeStruct((2, NUM_FEATURES), jnp.float32),
    )(x)
    return pl.pallas_call(
        _bucket_body,
        grid=(NBM,),
        in_specs=[
            pl.BlockSpec(memory_space=pltpu.SMEM),
            pl.BlockSpec((2, NUM_FEATURES), lambda i: (0, 0)),
            pl.BlockSpec((BM, NUM_FEATURES), lambda i: (i, 0)),
        ],
        out_specs=pl.BlockSpec((FPAD, BM), lambda i: (0, i)),
        out_shape=jax.ShapeDtypeStruct((FPAD, BATCH), jnp.int32),
    )(b_sig, mm, x)


def _gather_body(tt_hbm, bkt_hbm, out_hbm, tt_vm, bkt_vm, buf_vm, sem):
    c = lax.axis_index("c")
    s = lax.axis_index("s")
    w = s * NC + c
    g_lo = w * (NBLOCKS // NW)
    g_hi = g_lo + NBLOCKS // NW
    f_lo = g_lo // (TI * NBLK)
    f_hi = (g_hi - 1) // (TI * NBLK)

    pltpu.sync_copy(tt_hbm, tt_vm)

    def f_body(f, cnt):
        pltpu.sync_copy(bkt_hbm.at[f // 8, :, lax.rem(f, 8), :], bkt_vm)
        b_lo = jnp.maximum(g_lo - f * (TI * NBLK), 0)
        b_hi = jnp.minimum(g_hi - f * (TI * NBLK), TI * NBLK)

        def blk_body(bg, cnt3):
            ti = bg // NBLK
            tb = lax.rem(bg, NBLK)
            base0 = f * (EMBED_DIM * NUM_BUCKETS) + ti * (8 * NUM_BUCKETS)
            slot = lax.rem(cnt3, 2)
            dst = out_hbm.at[f, ti, pl.ds(tb * TJB, TJB)]

            @pl.when(cnt3 >= 2)
            def _():
                pltpu.make_async_copy(dst, buf_vm.at[slot], sem).wait()

            @plsc.parallel_loop(0, TJB, unroll=2)
            def tjl_body(tjl):
                tj = tb * TJB + tjl
                for jj in range(8):
                    bv = bkt_vm[tj, pl.ds(jj * 16, LANES)]
                    for d_in in range(8):
                        idx = bv + (base0 + d_in * NUM_BUCKETS)
                        buf_vm[slot, tjl, d_in, pl.ds(jj * 16, LANES)] = (
                            plsc.load_gather(tt_vm, [idx]))

            pltpu.async_copy(buf_vm.at[slot], dst, sem)
            return cnt3 + 1

        return lax.fori_loop(b_lo, b_hi, blk_body, cnt)

    cnt = lax.fori_loop(f_lo, f_hi + 1, f_body, 0)

    @pl.when(cnt >= 2)
    def _():
        pltpu.make_async_copy(out_hbm.at[0, 0, pl.ds(0, TJB)],
                              buf_vm.at[0], sem).wait()

    @pl.when(cnt >= 1)
    def _():
        pltpu.make_async_copy(out_hbm.at[0, 0, pl.ds(0, TJB)],
                              buf_vm.at[0], sem).wait()


@functools.cache
def _sc_gather():
    return functools.partial(
        pl.kernel,
        out_type=jax.ShapeDtypeStruct(
            (NUM_FEATURES, TI, TJ, 8, 128), jnp.float32),
        mesh=plsc.VectorSubcoreMesh(
            core_axis_name="c", subcore_axis_name="s", num_cores=NC, num_subcores=NS
        ),
        scratch_types=[
            pltpu.VMEM((TBL,), jnp.float32),
            pltpu.VMEM((TJ, 128), jnp.int32),
            pltpu.VMEM((2, TJB, 8, 128), jnp.float32),
            pltpu.SemaphoreType.DMA,
        ],
        compiler_params=pltpu.CompilerParams(
            use_tc_tiling_on_sc=False, needs_layout_passes=False
        ),
    )(_gather_body)


def kernel(x, boundaries, emb_tables):
    b_sig = jax.nn.sigmoid(boundaries)
    bkt_t = _bucket_t(x, b_sig)
    bkt4 = jnp.transpose(
        bkt_t.reshape(FPAD // 8, 8, TJ, 128), (0, 2, 1, 3))
    tt = jnp.transpose(emb_tables, (0, 2, 1)).reshape(TBL)
    out5 = _sc_gather()(tt, bkt4)
    o = jnp.transpose(out5, (2, 4, 0, 1, 3))
    return o.reshape(BATCH, NUM_FEATURES, EMBED_DIM)


# speedup vs baseline: 1.1118x; 1.1118x over previous
import functools

import jax
import jax.numpy as jnp
from jax import lax
from jax.experimental import pallas as pl
from jax.experimental.pallas import tpu as pltpu
from jax.experimental.pallas import tpu_sc as plsc

BATCH = 16384
NUM_FEATURES = 100
NUM_BUCKETS = 20
EMBED_DIM = 32

NC = 2
NS = 16
NW = NC * NS

TBL = NUM_FEATURES * EMBED_DIM * NUM_BUCKETS
FROW = EMBED_DIM * NUM_BUCKETS
NFW = 5
TBLP = (NUM_FEATURES + NFW) * FROW
LANES = 16

FPAD = 104
TI = EMBED_DIM // 8
TJ = BATCH // 128
TJB = 32
NBLK = TJ // TJB
NGROUPS = NUM_FEATURES * TI
NBLOCKS = NGROUPS * NBLK


def _bucket_body(bs_ref, x_ref, out_ref):
    x = x_ref[...]
    xmin = jnp.min(x, axis=0, keepdims=True)
    xmax = jnp.max(x, axis=0, keepdims=True)
    xn = (x - xmin) / (xmax - xmin + 1e-8)
    cnt = (xn > bs_ref[0]).astype(jnp.int32)
    for i in range(1, NUM_BUCKETS - 1):
        cnt += (xn > bs_ref[i]).astype(jnp.int32)
    bt = jnp.transpose(cnt)
    pad = jnp.zeros((FPAD - NUM_FEATURES, BATCH), jnp.int32)
    out_ref[...] = jnp.concatenate([bt, pad], axis=0)


def _bucket_t(x, b_sig):
    return pl.pallas_call(
        _bucket_body,
        in_specs=[
            pl.BlockSpec(memory_space=pltpu.SMEM),
            pl.BlockSpec(memory_space=pltpu.VMEM),
        ],
        out_specs=pl.BlockSpec(memory_space=pltpu.VMEM),
        out_shape=jax.ShapeDtypeStruct((FPAD, BATCH), jnp.int32),
    )(b_sig, x)


def _gather_body(tt_hbm, bkt_hbm, out_hbm, tt_vm, bkt_vm, buf_vm, sem):
    c = lax.axis_index("c")
    s = lax.axis_index("s")
    w = s * NC + c
    g_lo = w * (NBLOCKS // NW)
    g_hi = g_lo + NBLOCKS // NW
    f_lo = g_lo // (TI * NBLK)
    f_hi = (g_hi - 1) // (TI * NBLK)

    pltpu.sync_copy(tt_hbm.at[pl.ds(f_lo * FROW, NFW * FROW)], tt_vm)

    def f_body(f, cnt):
        pltpu.sync_copy(bkt_hbm.at[f // 8, :, lax.rem(f, 8), :], bkt_vm)
        b_lo = jnp.maximum(g_lo - f * (TI * NBLK), 0)
        b_hi = jnp.minimum(g_hi - f * (TI * NBLK), TI * NBLK)

        def blk_body(bg, cnt3):
            ti = bg // NBLK
            tb = lax.rem(bg, NBLK)
            base0 = (f - f_lo) * FROW + ti * (8 * NUM_BUCKETS)
            slot = lax.rem(cnt3, 2)
            dst = out_hbm.at[f, ti, pl.ds(tb * TJB, TJB)]

            @pl.when(cnt3 >= 2)
            def _():
                pltpu.make_async_copy(dst, buf_vm.at[slot], sem).wait()

            @plsc.parallel_loop(0, TJB, unroll=2)
            def tjl_body(tjl):
                tj = tb * TJB + tjl
                for jj in range(8):
                    bv = bkt_vm[tj, pl.ds(jj * 16, LANES)]
                    for d_in in range(8):
                        idx = bv + (base0 + d_in * NUM_BUCKETS)
                        buf_vm[slot, tjl, d_in, pl.ds(jj * 16, LANES)] = (
                            plsc.load_gather(tt_vm, [idx]))

            pltpu.async_copy(buf_vm.at[slot], dst, sem)
            return cnt3 + 1

        return lax.fori_loop(b_lo, b_hi, blk_body, cnt)

    cnt = lax.fori_loop(f_lo, f_hi + 1, f_body, 0)

    @pl.when(cnt >= 2)
    def _():
        pltpu.make_async_copy(out_hbm.at[0, 0, pl.ds(0, TJB)],
                              buf_vm.at[0], sem).wait()

    @pl.when(cnt >= 1)
    def _():
        pltpu.make_async_copy(out_hbm.at[0, 0, pl.ds(0, TJB)],
                              buf_vm.at[0], sem).wait()


@functools.cache
def _sc_gather():
    return functools.partial(
        pl.kernel,
        out_type=jax.ShapeDtypeStruct(
            (NUM_FEATURES, TI, TJ, 8, 128), jnp.float32),
        mesh=plsc.VectorSubcoreMesh(
            core_axis_name="c", subcore_axis_name="s", num_cores=NC, num_subcores=NS
        ),
        scratch_types=[
            pltpu.VMEM((NFW * FROW,), jnp.float32),
            pltpu.VMEM((TJ, 128), jnp.int32),
            pltpu.VMEM((2, TJB, 8, 128), jnp.float32),
            pltpu.SemaphoreType.DMA,
        ],
        compiler_params=pltpu.CompilerParams(
            use_tc_tiling_on_sc=False, needs_layout_passes=False
        ),
    )(_gather_body)


def kernel(x, boundaries, emb_tables):
    b_sig = jax.nn.sigmoid(boundaries)
    bkt_t = _bucket_t(x, b_sig)
    bkt4 = jnp.transpose(
        bkt_t.reshape(FPAD // 8, 8, TJ, 128), (0, 2, 1, 3))
    tt = jnp.transpose(emb_tables, (0, 2, 1)).reshape(TBL)
    tt = jnp.pad(tt, (0, TBLP - TBL))
    out5 = _sc_gather()(tt, bkt4)
    o = jnp.transpose(out5, (2, 4, 0, 1, 3))
    return o.reshape(BATCH, NUM_FEATURES, EMBED_DIM)


# speedup vs baseline: 1.1150x; 1.0028x over previous
import functools

import jax
import jax.numpy as jnp
from jax import lax
from jax.experimental import pallas as pl
from jax.experimental.pallas import tpu as pltpu
from jax.experimental.pallas import tpu_sc as plsc

BATCH = 16384
NUM_FEATURES = 100
NUM_BUCKETS = 20
EMBED_DIM = 32

NC = 2
NS = 16
NW = NC * NS

TBL = NUM_FEATURES * EMBED_DIM * NUM_BUCKETS
FROW = EMBED_DIM * NUM_BUCKETS
NFW = 5
TBLP = (NUM_FEATURES + NFW) * FROW
LANES = 16

FPAD = 104
TI = EMBED_DIM // 8
TJ = BATCH // 128
TJB = 32
NBLK = TJ // TJB
NGROUPS = NUM_FEATURES * TI
NBLOCKS = NGROUPS * NBLK


def _bucket_body(bs_ref, x_ref, out_ref):
    x = x_ref[...]
    xmin = jnp.min(x, axis=0, keepdims=True)
    xmax = jnp.max(x, axis=0, keepdims=True)
    xn = (x - xmin) / (xmax - xmin + 1e-8)
    cnt = (xn > bs_ref[0]).astype(jnp.int32)
    for i in range(1, NUM_BUCKETS - 1):
        cnt += (xn > bs_ref[i]).astype(jnp.int32)
    bt = jnp.transpose(cnt)
    pad = jnp.zeros((FPAD - NUM_FEATURES, BATCH), jnp.int32)
    out_ref[...] = jnp.concatenate([bt, pad], axis=0)


def _bucket_t(x, b_sig):
    return pl.pallas_call(
        _bucket_body,
        in_specs=[
            pl.BlockSpec(memory_space=pltpu.SMEM),
            pl.BlockSpec(memory_space=pltpu.VMEM),
        ],
        out_specs=pl.BlockSpec(memory_space=pltpu.VMEM),
        out_shape=jax.ShapeDtypeStruct((FPAD, BATCH), jnp.int32),
    )(b_sig, x)


def _gather_body(tt_hbm, bkt_hbm, out_hbm, tt_vm, bkt_vm, buf_vm, sem):
    c = lax.axis_index("c")
    s = lax.axis_index("s")
    w = s * NC + c
    g_lo = w * (NBLOCKS // NW)
    g_hi = g_lo + NBLOCKS // NW
    f_lo = g_lo // (TI * NBLK)
    f_hi = (g_hi - 1) // (TI * NBLK)

    pltpu.sync_copy(tt_hbm.at[pl.ds(f_lo * FROW, NFW * FROW)], tt_vm)

    def f_body(f, cnt):
        pltpu.sync_copy(bkt_hbm.at[f // 8, :, lax.rem(f, 8), :], bkt_vm)
        b_lo = jnp.maximum(g_lo - f * (TI * NBLK), 0)
        b_hi = jnp.minimum(g_hi - f * (TI * NBLK), TI * NBLK)

        def blk_body(bg, cnt3):
            ti = bg // NBLK
            tb = lax.rem(bg, NBLK)
            base0 = (f - f_lo) * FROW + ti * (8 * NUM_BUCKETS)
            slot = lax.rem(cnt3, 2)
            dst = out_hbm.at[f, ti, pl.ds(tb * TJB, TJB)]

            @pl.when(cnt3 >= 2)
            def _():
                pltpu.make_async_copy(dst, buf_vm.at[slot], sem).wait()

            @plsc.parallel_loop(0, TJB, unroll=4)
            def tjl_body(tjl):
                tj = tb * TJB + tjl
                for jj in range(8):
                    bv = bkt_vm[tj, pl.ds(jj * 16, LANES)]
                    for d_in in range(8):
                        idx = bv + (base0 + d_in * NUM_BUCKETS)
                        buf_vm[slot, tjl, d_in, pl.ds(jj * 16, LANES)] = (
                            plsc.load_gather(tt_vm, [idx]))

            pltpu.async_copy(buf_vm.at[slot], dst, sem)
            return cnt3 + 1

        return lax.fori_loop(b_lo, b_hi, blk_body, cnt)

    cnt = lax.fori_loop(f_lo, f_hi + 1, f_body, 0)

    @pl.when(cnt >= 2)
    def _():
        pltpu.make_async_copy(out_hbm.at[0, 0, pl.ds(0, TJB)],
                              buf_vm.at[0], sem).wait()

    @pl.when(cnt >= 1)
    def _():
        pltpu.make_async_copy(out_hbm.at[0, 0, pl.ds(0, TJB)],
                              buf_vm.at[0], sem).wait()


@functools.cache
def _sc_gather():
    return functools.partial(
        pl.kernel,
        out_type=jax.ShapeDtypeStruct(
            (NUM_FEATURES, TI, TJ, 8, 128), jnp.float32),
        mesh=plsc.VectorSubcoreMesh(
            core_axis_name="c", subcore_axis_name="s", num_cores=NC, num_subcores=NS
        ),
        scratch_types=[
            pltpu.VMEM((NFW * FROW,), jnp.float32),
            pltpu.VMEM((TJ, 128), jnp.int32),
            pltpu.VMEM((2, TJB, 8, 128), jnp.float32),
            pltpu.SemaphoreType.DMA,
        ],
        compiler_params=pltpu.CompilerParams(
            use_tc_tiling_on_sc=False, needs_layout_passes=False
        ),
    )(_gather_body)


def kernel(x, boundaries, emb_tables):
    b_sig = jax.nn.sigmoid(boundaries)
    bkt_t = _bucket_t(x, b_sig)
    bkt4 = jnp.transpose(
        bkt_t.reshape(FPAD // 8, 8, TJ, 128), (0, 2, 1, 3))
    tt = jnp.transpose(emb_tables, (0, 2, 1)).reshape(TBL)
    tt = jnp.pad(tt, (0, TBLP - TBL))
    out5 = _sc_gather()(tt, bkt4)
    o = jnp.transpose(out5, (2, 4, 0, 1, 3))
    return o.reshape(BATCH, NUM_FEATURES, EMBED_DIM)
